# Initial kernel scaffold; baseline (speedup 1.0000x reference)
import jax
import jax.numpy as jnp
from jax.experimental import pallas as pl


def kernel(x, edge_index, W0, b0, W1, b1, W2, b2):
    raise NotImplementedError("write your pallas kernel here")


# speedup vs baseline: 18.8257x; 18.8257x over previous
import functools

import jax
import jax.numpy as jnp
from jax import lax
from jax.experimental import pallas as pl
from jax.experimental.pallas import tpu as pltpu
from jax.experimental.pallas import tpu_sc as plsc

NC = 2
NS = 16
NW = NC * NS
ECHUNK = 128

_MESH = plsc.VectorSubcoreMesh(
    core_axis_name="c", subcore_axis_name="s", num_cores=NC, num_subcores=NS
)


def _deg_call(dstp, zeros1d):
    nw, ch, _ = dstp.shape
    npad = zeros1d.shape[0]

    @functools.partial(
        pl.kernel,
        out_type=jax.ShapeDtypeStruct((NC, npad), jnp.float32),
        mesh=_MESH,
        scratch_types=[
            pltpu.VMEM((ch, ECHUNK), jnp.int32),
            pltpu.VMEM((ECHUNK,), jnp.float32),
            pltpu.VMEM_SHARED((npad,), jnp.float32),
            pltpu.SemaphoreType.DMA,
        ],
    )
    def deg_kernel(dst_hbm, zeros_hbm, deg_out, dst_v, ones_v, shared_deg, sem):
        c = lax.axis_index("c")
        s = lax.axis_index("s")
        wid = c * NS + s
        pltpu.sync_copy(dst_hbm.at[wid], dst_v)
        for i in range(ECHUNK // 16):
            ones_v[pl.ds(i * 16, 16)] = jnp.ones((16,), jnp.float32)

        @pl.when(s == 0)
        def _():
            pltpu.sync_copy(zeros_hbm, shared_deg)

        plsc.subcore_barrier()

        def body(j, carry):
            pltpu.sync_copy(ones_v, shared_deg.at[dst_v.at[j]], add=True)
            return carry

        lax.fori_loop(0, ch, body, 0)
        plsc.subcore_barrier()

        @pl.when(s == 0)
        def _():
            pltpu.sync_copy(shared_deg, deg_out.at[c])

    return deg_kernel(dstp, zeros1d)


def _scatter_call(y, srcp, dstp, zeros2d):
    nw, ch, _ = srcp.shape
    npad, d = zeros2d.shape
    stride = npad // NS

    @functools.partial(
        pl.kernel,
        out_type=jax.ShapeDtypeStruct((NC, npad, d), jnp.float32),
        mesh=_MESH,
        scratch_types=[
            pltpu.VMEM((ch, ECHUNK), jnp.int32),
            pltpu.VMEM((ch, ECHUNK), jnp.int32),
            pltpu.VMEM((ECHUNK, d), jnp.float32),
            pltpu.VMEM_SHARED((npad, d), jnp.float32),
            pltpu.SemaphoreType.DMA,
        ],
    )
    def scat_kernel(y_hbm, src_hbm, dst_hbm, zeros_hbm, z_out,
                    src_v, dst_v, buf, shared_z, sem):
        c = lax.axis_index("c")
        s = lax.axis_index("s")
        wid = c * NS + s
        pltpu.sync_copy(src_hbm.at[wid], src_v)
        pltpu.sync_copy(dst_hbm.at[wid], dst_v)

        @pl.when(s == 0)
        def _():
            pltpu.sync_copy(zeros_hbm, shared_z)

        plsc.subcore_barrier()

        def body(j, carry):
            pltpu.async_copy(y_hbm.at[src_v.at[j]], buf, sem).wait()
            pltpu.sync_copy(buf, shared_z.at[dst_v.at[j]], add=True)
            return carry

        lax.fori_loop(0, ch, body, 0)
        plsc.subcore_barrier()
        pltpu.sync_copy(
            shared_z.at[pl.ds(s * stride, stride)],
            z_out.at[c, pl.ds(s * stride, stride)],
        )

    return scat_kernel(y, srcp, dstp, zeros2d)


def _tc_first(degT, x, w0):
    n, _ = x.shape
    d_out = w0.shape[1]
    npad = degT.shape[0]

    def body(deg_ref, x_ref, w_ref, dinv_ref, y_ref):
        cnt = deg_ref[:, 0:1] + deg_ref[:, 1:2] + 1.0
        dinv = lax.rsqrt(cnt)
        dinv_ref[...] = dinv
        xw = jnp.dot(x_ref[...], w_ref[...], preferred_element_type=jnp.float32)
        y_ref[...] = xw * dinv[:n]

    return pl.pallas_call(
        body,
        out_shape=(
            jax.ShapeDtypeStruct((npad, 1), jnp.float32),
            jax.ShapeDtypeStruct((n, d_out), jnp.float32),
        ),
    )(degT, x, w0)


def _tc_mid(z, y, dinv, b, hprev, wn):
    n, d = y.shape

    def body(z_ref, y_ref, dinv_ref, b_ref, *rest):
        if hprev is not None:
            h_ref, w_ref, hout_ref, yout_ref = rest
        else:
            w_ref, hout_ref, yout_ref = rest
        dv = dinv_ref[:n]
        g = (z_ref[0, :n] + z_ref[1, :n] + y_ref[...]) * dv + b_ref[...]
        if hprev is not None:
            g = g + h_ref[...]
        h = jnp.where(g > 0.0, g, jnp.exp(g) - 1.0)
        hout_ref[...] = h
        yout_ref[...] = (
            jnp.dot(h, w_ref[...], preferred_element_type=jnp.float32) * dv
        )

    args = [z, y, dinv, b] + ([hprev] if hprev is not None else []) + [wn]
    return pl.pallas_call(
        body,
        out_shape=(
            jax.ShapeDtypeStruct((n, d), jnp.float32),
            jax.ShapeDtypeStruct((n, wn.shape[1]), jnp.float32),
        ),
    )(*args)


def _tc_last(z, y, dinv, b, hprev):
    n, d = y.shape

    def body(z_ref, y_ref, dinv_ref, b_ref, h_ref, hout_ref):
        g = (z_ref[0, :n] + z_ref[1, :n] + y_ref[...]) * dinv_ref[:n] + b_ref[...]
        g = g + h_ref[...]
        hout_ref[...] = jnp.where(g > 0.0, g, jnp.exp(g) - 1.0)

    return pl.pallas_call(
        body,
        out_shape=jax.ShapeDtypeStruct((n, d), jnp.float32),
    )(z, y, dinv, b, hprev)


def kernel(x, edge_index, W0, b0, W1, b1, W2, b2):
    n, d = x.shape
    e = edge_index.shape[1]
    npad = -(-(n + 1) // (NS * 8)) * (NS * 8)
    ch = -(-e // (NW * ECHUNK))
    epad = NW * ch * ECHUNK
    pad = epad - e

    src = edge_index[0].astype(jnp.int32)
    dst = edge_index[1].astype(jnp.int32)
    pad_i = jnp.arange(pad, dtype=jnp.int32)
    srcp = jnp.concatenate([src, pad_i % n]).reshape(NW, ch, ECHUNK)
    dstp = jnp.concatenate([dst, n + pad_i % (npad - n)]).reshape(
        NW, ch, ECHUNK
    )
    zeros2d = jnp.zeros((npad, d), jnp.float32)
    zeros1d = jnp.zeros((npad,), jnp.float32)

    deg = _deg_call(dstp, zeros1d)
    dinv, y = _tc_first(deg.T, x, W0)

    z = _scatter_call(y, srcp, dstp, zeros2d)
    h, y = _tc_mid(z, y, dinv, b0, None, W1)

    z = _scatter_call(y, srcp, dstp, zeros2d)
    h, y = _tc_mid(z, y, dinv, b1, h, W2)

    z = _scatter_call(y, srcp, dstp, zeros2d)
    return _tc_last(z, y, dinv, b2, h)


# speedup vs baseline: 27.2628x; 1.4482x over previous
import functools

import jax
import jax.numpy as jnp
from jax import lax
from jax.experimental import pallas as pl
from jax.experimental.pallas import tpu as pltpu
from jax.experimental.pallas import tpu_sc as plsc

NC = 2
NS = 16
NW = NC * NS
ECHUNK = 128

_MESH = plsc.VectorSubcoreMesh(
    core_axis_name="c", subcore_axis_name="s", num_cores=NC, num_subcores=NS
)


def _deg_call(dstp, zeros1d):
    nw, ch, _ = dstp.shape
    npad = zeros1d.shape[0]

    @functools.partial(
        pl.kernel,
        out_type=jax.ShapeDtypeStruct((NC, npad), jnp.float32),
        mesh=_MESH,
        scratch_types=[
            pltpu.VMEM((ch, ECHUNK), jnp.int32),
            pltpu.VMEM((ECHUNK,), jnp.float32),
            pltpu.VMEM_SHARED((npad,), jnp.float32),
            pltpu.SemaphoreType.DMA,
        ],
    )
    def deg_kernel(dst_hbm, zeros_hbm, deg_out, dst_v, ones_v, shared_deg, sem):
        c = lax.axis_index("c")
        s = lax.axis_index("s")
        wid = c * NS + s
        pltpu.sync_copy(dst_hbm.at[wid], dst_v)
        for i in range(ECHUNK // 16):
            ones_v[pl.ds(i * 16, 16)] = jnp.ones((16,), jnp.float32)

        @pl.when(s == 0)
        def _():
            pltpu.sync_copy(zeros_hbm, shared_deg)

        plsc.subcore_barrier()

        def body(j, carry):
            pltpu.sync_copy(ones_v, shared_deg.at[dst_v.at[j]], add=True)
            return carry

        lax.fori_loop(0, ch, body, 0)
        plsc.subcore_barrier()

        @pl.when(s == 0)
        def _():
            pltpu.sync_copy(shared_deg, deg_out.at[c])

    return deg_kernel(dstp, zeros1d)


def _scatter_call(y, srcp, dstp, zeros2d):
    nw, ch, _ = srcp.shape
    npad, d = zeros2d.shape
    stride = npad // NS
    nph = 2
    chp = ch // nph

    @functools.partial(
        pl.kernel,
        out_type=jax.ShapeDtypeStruct((NC, npad, d), jnp.float32),
        mesh=_MESH,
        scratch_types=[
            pltpu.VMEM((chp, ECHUNK), jnp.int32),
            pltpu.VMEM((chp, ECHUNK), jnp.int32),
            pltpu.VMEM((ECHUNK, d), jnp.float32),
            pltpu.VMEM((ECHUNK, d), jnp.float32),
            pltpu.VMEM_SHARED((npad, d), jnp.float32),
            pltpu.SemaphoreType.DMA,
            pltpu.SemaphoreType.DMA,
        ],
    )
    def scat_kernel(y_hbm, src_hbm, dst_hbm, zeros_hbm, z_out,
                    src_v, dst_v, buf0, buf1, shared_z, sem0, sem1):
        c = lax.axis_index("c")
        s = lax.axis_index("s")
        wid = c * NS + s

        @pl.when(s == 0)
        def _():
            pltpu.sync_copy(zeros_hbm, shared_z)

        plsc.subcore_barrier()

        for ph in range(nph):
            pltpu.sync_copy(src_hbm.at[wid, pl.ds(ph * chp, chp)], src_v)
            pltpu.sync_copy(dst_hbm.at[wid, pl.ds(ph * chp, chp)], dst_v)
            pltpu.async_copy(y_hbm.at[src_v.at[0]], buf0, sem0)

            def body(k2, carry):
                j0 = 2 * k2
                pltpu.async_copy(y_hbm.at[src_v.at[j0 + 1]], buf1, sem1)
                pltpu.make_async_copy(y_hbm.at[src_v.at[j0]], buf0, sem0).wait()
                pltpu.sync_copy(buf0, shared_z.at[dst_v.at[j0]], add=True)

                @pl.when(j0 + 2 < chp)
                def _():
                    pltpu.async_copy(y_hbm.at[src_v.at[j0 + 2]], buf0, sem0)

                pltpu.make_async_copy(y_hbm.at[src_v.at[j0 + 1]], buf1, sem1).wait()
                pltpu.sync_copy(buf1, shared_z.at[dst_v.at[j0 + 1]], add=True)
                return carry

            lax.fori_loop(0, chp // 2, body, 0)
        plsc.subcore_barrier()
        pltpu.sync_copy(
            shared_z.at[pl.ds(s * stride, stride)],
            z_out.at[c, pl.ds(s * stride, stride)],
        )

    return scat_kernel(y, srcp, dstp, zeros2d)


def _tc_first(degT, x, w0):
    n, _ = x.shape
    d_out = w0.shape[1]
    npad = degT.shape[0]

    def body(deg_ref, x_ref, w_ref, dinv_ref, y_ref):
        cnt = deg_ref[:, 0:1] + deg_ref[:, 1:2] + 1.0
        dinv = lax.rsqrt(cnt)
        dinv_ref[...] = dinv
        xw = jnp.dot(x_ref[...], w_ref[...], preferred_element_type=jnp.float32)
        y_ref[...] = xw * dinv[:n]

    return pl.pallas_call(
        body,
        out_shape=(
            jax.ShapeDtypeStruct((npad, 1), jnp.float32),
            jax.ShapeDtypeStruct((n, d_out), jnp.float32),
        ),
    )(degT, x, w0)


def _tc_mid(z, y, dinv, b, hprev, wn):
    n, d = y.shape

    def body(z_ref, y_ref, dinv_ref, b_ref, *rest):
        if hprev is not None:
            h_ref, w_ref, hout_ref, yout_ref = rest
        else:
            w_ref, hout_ref, yout_ref = rest
        dv = dinv_ref[:n]
        g = (z_ref[0, :n] + z_ref[1, :n] + y_ref[...]) * dv + b_ref[...]
        if hprev is not None:
            g = g + h_ref[...]
        h = jnp.where(g > 0.0, g, jnp.exp(g) - 1.0)
        hout_ref[...] = h
        yout_ref[...] = (
            jnp.dot(h, w_ref[...], preferred_element_type=jnp.float32) * dv
        )

    args = [z, y, dinv, b] + ([hprev] if hprev is not None else []) + [wn]
    return pl.pallas_call(
        body,
        out_shape=(
            jax.ShapeDtypeStruct((n, d), jnp.float32),
            jax.ShapeDtypeStruct((n, wn.shape[1]), jnp.float32),
        ),
    )(*args)


def _tc_last(z, y, dinv, b, hprev):
    n, d = y.shape

    def body(z_ref, y_ref, dinv_ref, b_ref, h_ref, hout_ref):
        g = (z_ref[0, :n] + z_ref[1, :n] + y_ref[...]) * dinv_ref[:n] + b_ref[...]
        g = g + h_ref[...]
        hout_ref[...] = jnp.where(g > 0.0, g, jnp.exp(g) - 1.0)

    return pl.pallas_call(
        body,
        out_shape=jax.ShapeDtypeStruct((n, d), jnp.float32),
    )(z, y, dinv, b, hprev)


def kernel(x, edge_index, W0, b0, W1, b1, W2, b2):
    n, d = x.shape
    e = edge_index.shape[1]
    npad = -(-(n + 1) // (NS * 8)) * (NS * 8)
    ch = -(-e // (NW * ECHUNK))
    ch = -(-ch // 4) * 4
    epad = NW * ch * ECHUNK
    pad = epad - e

    src = edge_index[0].astype(jnp.int32)
    dst = edge_index[1].astype(jnp.int32)
    pad_i = jnp.arange(pad, dtype=jnp.int32)
    srcp = jnp.concatenate([src, pad_i % n]).reshape(NW, ch, ECHUNK)
    dstp = jnp.concatenate([dst, n + pad_i % (npad - n)]).reshape(
        NW, ch, ECHUNK
    )
    zeros2d = jnp.zeros((npad, d), jnp.float32)
    zeros1d = jnp.zeros((npad,), jnp.float32)

    deg = _deg_call(dstp, zeros1d)
    dinv, y = _tc_first(deg.T, x, W0)

    z = _scatter_call(y, srcp, dstp, zeros2d)
    h, y = _tc_mid(z, y, dinv, b0, None, W1)

    z = _scatter_call(y, srcp, dstp, zeros2d)
    h, y = _tc_mid(z, y, dinv, b1, h, W2)

    z = _scatter_call(y, srcp, dstp, zeros2d)
    return _tc_last(z, y, dinv, b2, h)
